# baseline (speedup 1.0000x reference)
import jax
import jax.numpy as jnp
from jax.experimental import pallas as pl

ATOM_FDIM = 128
BOND_FDIM = 6
H = 64
KHEAD = 2
DEPTH = 2
N = 64
NBS = 8
BM = 16

_F32 = jnp.float32


def _lrelu(x, s):
    return jnp.maximum(x, s * x)


def _bdot(a, b):
    return jax.lax.dot_general(
        a, b, (((2,), (1,)), ((0,), (0,))), preferred_element_type=_F32)


def _prep_params(params):
    out = []

    def lin(p):
        W, b = p
        out.append(W.T)
        out.append(b.reshape(1, -1))

    lin(params["vertex_embedding"])
    for it in range(DEPTH):
        U2, U2b = params["label_U2"][it]
        for k in range(KHEAD):
            lin(params["W_a_main"][it][k])
            lin(params["W_main"][it][k])
        out.append(U2[:, :H].T)
        sf_w = jnp.concatenate(
            [params["W_a_super"][it][0][0].T, params["W_a_super"][it][1][0].T,
             params["W_super_to_main"][it][0].T, params["W_super"][it][0].T],
            axis=1)
        sf_b = jnp.concatenate(
            [params["W_a_super"][it][0][1], params["W_a_super"][it][1][1],
             params["W_super_to_main"][it][1], params["W_super"][it][1]]
        ).reshape(1, 4 * H)
        out.append(sf_w)
        out.append(sf_b)
        out.append(jnp.concatenate(
            [params["W_bmm"][it][0][0].reshape(1, H),
             params["W_bmm"][it][1][0].reshape(1, H)], axis=1))
        out.append(jnp.concatenate(
            [jnp.broadcast_to(params["W_bmm"][it][0][1].reshape(1, 1) / H, (1, H)),
             jnp.broadcast_to(params["W_bmm"][it][1][1].reshape(1, 1) / H, (1, H))],
            axis=1))
        out.append(U2[:, H:].T)
        out.append(U2b.reshape(1, -1))
        U1, U1b = params["label_U1"][it]
        out.append(U1[:, :H].T)
        out.append(U1[:, H:].T)
        out.append(U1b.reshape(1, -1))
        m2sWf, m2sbf = params["W_main_to_super"][it]
        out.append(m2sWf[:, :H].T)
        out.append(m2sWf[:, H:].T)
        out.append(m2sbf.reshape(1, -1))
        lin(params["W_zm1"][it])
        lin(params["W_zm2"][it])
        lin(params["W_zs1"][it])
        lin(params["W_zs2"][it])
    return out


def _body(*refs):
    (af_ref, bf_ref, anb_ref, bnb_ref, nmask_ref, dmask_ref, pat_ref) = refs[:7]
    vf_out, sf_out = refs[-2:]
    prefs = list(refs[7:-2])

    def nxt():
        return prefs.pop(0)[...]

    veW, veb = nxt(), nxt()
    depth_p = [tuple(nxt() for _ in range(29)) for _ in range(DEPTH)]

    AF = af_ref[...].reshape(BM * N, ATOM_FDIM)
    bf = bf_ref[...].reshape(BM * N, BOND_FDIM)
    anb = anb_ref[...]
    bnb = bnb_ref[...]
    nmask_row = nmask_ref[...]
    dmask3 = dmask_ref[...]
    dmask_row = dmask3.reshape(BM, N)

    vf = _lrelu(jnp.dot(AF, veW, preferred_element_type=_F32) + veb, 0.01)
    sf = _bdot(dmask3, vf.reshape(BM, N, H)).reshape(BM, H)

    iota2 = jax.lax.broadcasted_iota(jnp.int32, (BM, N * NBS, 2 * N), 2)
    target = jnp.where(iota2 < N, anb, bnb)
    oh = (target == (iota2 & (N - 1))).astype(jnp.bfloat16)

    R = (pat_ref[...] * nmask_row).astype(jnp.bfloat16)

    for it in range(DEPTH):
        (am0W, am0b, m0W, m0b, am1W, am1b, m1W, m1b, U2v, sfW, sfb,
         bmm_w, bmm_bd, U2e, U2b, U1v, U1n, U1b,
         m2sW0, m2sW1, m2sb, zm1W, zm1b, zm2W, zm2b, zs1W, zs1b, zs2W, zs2b
         ) = depth_p[it]
        T = jnp.tanh(jnp.dot(sf, sfW, preferred_element_type=_F32) + sfb)
        Cpair = (T[:, :2 * H] * bmm_w).reshape(BM, 2 * H)
        s2m = T[:, 2 * H:3 * H]
        ss = T[:, 3 * H:]
        heads = []
        for k in range(KHEAD):
            amW, amb = (am0W, am0b) if k == 0 else (am1W, am1b)
            mW, mb = (m0W, m0b) if k == 0 else (m1W, m1b)
            AM = jnp.tanh(jnp.dot(vf, amW, preferred_element_type=_F32) + amb)
            V = jnp.dot(vf, mW, preferred_element_type=_F32) + mb
            C = Cpair[:, k * H:(k + 1) * H].reshape(BM, 1, H)
            t = (AM.reshape(BM, N, H) * C
                 + bmm_bd[:, k * H:(k + 1) * H].reshape(1, 1, H))
            a = jnp.sum(t, axis=-1)
            amax = jnp.max(a, axis=-1, keepdims=True)
            aexp = jnp.exp(a - amax) * dmask_row
            attn = aexp / (jnp.sum(aexp, axis=-1, keepdims=True) + 1e-6)
            heads.append(_bdot(attn.reshape(BM, 1, N), V.reshape(BM, N, H)))
        h0 = heads[0].reshape(BM, H)
        h1 = heads[1].reshape(BM, H)
        mts = jnp.tanh(jnp.dot(h0, m2sW0, preferred_element_type=_F32)
                       + jnp.dot(h1, m2sW1, preferred_element_type=_F32) + m2sb)

        Pv = jnp.dot(vf, U2v, preferred_element_type=_F32)
        Pe = jnp.dot(bf, U2e, preferred_element_type=_F32)
        Pcat = jnp.concatenate(
            [Pv.reshape(BM, N, H), Pe.reshape(BM, N, H)], axis=1)
        G = _bdot(oh, Pcat)
        L = _lrelu(G + U2b.reshape(1, 1, H), 0.1)
        nei = _bdot(R, L).reshape(BM * N, H)
        main_self = _lrelu(
            jnp.dot(vf, U1v, preferred_element_type=_F32)
            + jnp.dot(nei, U1n, preferred_element_type=_F32) + U1b, 0.1)

        zm_in = (jnp.dot(main_self, zm1W, preferred_element_type=_F32) + zm1b
                 ).reshape(BM, N, H)
        zm_sup = jnp.dot(s2m, zm2W, preferred_element_type=_F32) + zm2b
        zm = jax.nn.sigmoid(zm_in + zm_sup.reshape(BM, 1, H))
        vf = ((1 - zm) * main_self.reshape(BM, N, H)
              + zm * s2m.reshape(BM, 1, H)).reshape(BM * N, H)
        zs = jax.nn.sigmoid(
            jnp.dot(ss, zs1W, preferred_element_type=_F32) + zs1b
            + jnp.dot(mts, zs2W, preferred_element_type=_F32) + zs2b)
        sf = (1 - zs) * ss + zs * mts

    vf_out[...] = vf.reshape(BM, N, H)
    sf_out[...] = sf.reshape(BM, 1, H)


def kernel(batch_size, atom_fea, bond_fea, d_anb, d_bnb, d_nbs_mask, d_mask, params):
    B = atom_fea.shape[0]
    flat = _prep_params(params)

    anb2 = d_anb.astype(jnp.int32).reshape(B, N * NBS, 1)
    bnb2 = d_bnb.astype(jnp.int32).reshape(B, N * NBS, 1)
    nmask2 = d_nbs_mask.reshape(B, 1, N * NBS)
    dmask2 = d_mask.reshape(B, 1, N)
    pat = jnp.repeat(jnp.eye(N, dtype=jnp.float32), NBS, axis=1
                     ).reshape(1, N, N * NBS)

    def rep(shape):
        nd = len(shape)
        return pl.BlockSpec(shape, lambda i, _n=nd: (0,) * _n)

    in_specs = [
        pl.BlockSpec((BM, N, ATOM_FDIM), lambda i: (i, 0, 0)),
        pl.BlockSpec((BM, N, BOND_FDIM), lambda i: (i, 0, 0)),
        pl.BlockSpec((BM, N * NBS, 1), lambda i: (i, 0, 0)),
        pl.BlockSpec((BM, N * NBS, 1), lambda i: (i, 0, 0)),
        pl.BlockSpec((BM, 1, N * NBS), lambda i: (i, 0, 0)),
        pl.BlockSpec((BM, 1, N), lambda i: (i, 0, 0)),
        rep((1, N, N * NBS)),
    ] + [rep(a.shape) for a in flat]

    out_shape = (
        jax.ShapeDtypeStruct((B, N, H), jnp.float32),
        jax.ShapeDtypeStruct((B, 1, H), jnp.float32),
    )
    out_specs = (
        pl.BlockSpec((BM, N, H), lambda i: (i, 0, 0)),
        pl.BlockSpec((BM, 1, H), lambda i: (i, 0, 0)),
    )
    vf, sf = pl.pallas_call(
        _body,
        grid=(B // BM,),
        in_specs=in_specs,
        out_specs=out_specs,
        out_shape=out_shape,
    )(atom_fea, bond_fea, anb2, bnb2, nmask2, dmask2, pat, *flat)
    return vf, sf


# speedup vs baseline: 1.1104x; 1.1104x over previous
import jax
import jax.numpy as jnp
from jax.experimental import pallas as pl

ATOM_FDIM = 128
BOND_FDIM = 6
H = 64
KHEAD = 2
DEPTH = 2
N = 64
NBS = 8
BM = 32

_F32 = jnp.float32


def _lrelu(x, s):
    return jnp.maximum(x, s * x)


def _bdot(a, b):
    return jax.lax.dot_general(
        a, b, (((2,), (1,)), ((0,), (0,))), preferred_element_type=_F32)


def _prep_params(params):
    out = []

    def lin(p):
        W, b = p
        out.append(W.T)
        out.append(b.reshape(1, -1))

    lin(params["vertex_embedding"])
    for it in range(DEPTH):
        U2, U2b = params["label_U2"][it]
        for k in range(KHEAD):
            lin(params["W_a_main"][it][k])
            lin(params["W_main"][it][k])
        out.append(U2[:, :H].T)
        sf_w = jnp.concatenate(
            [params["W_a_super"][it][0][0].T, params["W_a_super"][it][1][0].T,
             params["W_super_to_main"][it][0].T, params["W_super"][it][0].T],
            axis=1)
        sf_b = jnp.concatenate(
            [params["W_a_super"][it][0][1], params["W_a_super"][it][1][1],
             params["W_super_to_main"][it][1], params["W_super"][it][1]]
        ).reshape(1, 4 * H)
        out.append(sf_w)
        out.append(sf_b)
        out.append(jnp.concatenate(
            [params["W_bmm"][it][0][0].reshape(1, H),
             params["W_bmm"][it][1][0].reshape(1, H)], axis=1))
        out.append(jnp.concatenate(
            [jnp.broadcast_to(params["W_bmm"][it][0][1].reshape(1, 1) / H, (1, H)),
             jnp.broadcast_to(params["W_bmm"][it][1][1].reshape(1, 1) / H, (1, H))],
            axis=1))
        out.append(U2[:, H:].T)
        out.append(U2b.reshape(1, -1) / 2)
        U1, U1b = params["label_U1"][it]
        out.append(U1[:, :H].T)
        out.append(U1[:, H:].T)
        out.append(U1b.reshape(1, -1))
        m2sWf, m2sbf = params["W_main_to_super"][it]
        out.append(m2sWf[:, :H].T)
        out.append(m2sWf[:, H:].T)
        out.append(m2sbf.reshape(1, -1))
        lin(params["W_zm1"][it])
        lin(params["W_zm2"][it])
        lin(params["W_zs1"][it])
        lin(params["W_zs2"][it])
    return out


def _body(*refs):
    (af_ref, bf_ref, anb_ref, bnb_ref, nmask_ref, dmask_ref, pat_ref) = refs[:7]
    vf_out, sf_out = refs[-2:]
    prefs = list(refs[7:-2])

    def nxt():
        return prefs.pop(0)[...]

    veW, veb = nxt(), nxt()
    depth_p = [tuple(nxt() for _ in range(29)) for _ in range(DEPTH)]

    AF = af_ref[...].reshape(BM * N, ATOM_FDIM)
    bf = bf_ref[...].reshape(BM * N, BOND_FDIM)
    anb = anb_ref[...]
    bnb = bnb_ref[...]
    nmask_row = nmask_ref[...]
    dmask3 = dmask_ref[...]
    dmask_row = dmask3.reshape(BM, N)

    vf = _lrelu(jnp.dot(AF, veW, preferred_element_type=_F32) + veb, 0.01)
    sf = _bdot(dmask3, vf.reshape(BM, N, H)).reshape(BM, H)

    iota2 = jax.lax.broadcasted_iota(jnp.int32, (BM, N * NBS, 2 * N), 2)
    target = jnp.where(iota2 < N, anb, bnb)
    oh = (target == (iota2 & (N - 1))).astype(jnp.bfloat16)

    R = (pat_ref[...] * nmask_row).astype(jnp.bfloat16)

    for it in range(DEPTH):
        (am0W, am0b, m0W, m0b, am1W, am1b, m1W, m1b, U2v, sfW, sfb,
         bmm_w, bmm_bd, U2e, U2b, U1v, U1n, U1b,
         m2sW0, m2sW1, m2sb, zm1W, zm1b, zm2W, zm2b, zs1W, zs1b, zs2W, zs2b
         ) = depth_p[it]
        T = jnp.tanh(jnp.dot(sf, sfW, preferred_element_type=_F32) + sfb)
        Cpair = (T[:, :2 * H] * bmm_w).reshape(BM, 2 * H)
        s2m = T[:, 2 * H:3 * H]
        ss = T[:, 3 * H:]
        heads = []
        for k in range(KHEAD):
            amW, amb = (am0W, am0b) if k == 0 else (am1W, am1b)
            mW, mb = (m0W, m0b) if k == 0 else (m1W, m1b)
            AM = jnp.tanh(jnp.dot(vf, amW, preferred_element_type=_F32) + amb)
            V = jnp.dot(vf, mW, preferred_element_type=_F32) + mb
            C = Cpair[:, k * H:(k + 1) * H].reshape(BM, 1, H)
            t = (AM.reshape(BM, N, H) * C
                 + bmm_bd[:, k * H:(k + 1) * H].reshape(1, 1, H))
            a = jnp.sum(t, axis=-1)
            amax = jnp.max(a, axis=-1, keepdims=True)
            aexp = jnp.exp(a - amax) * dmask_row
            attn = aexp / (jnp.sum(aexp, axis=-1, keepdims=True) + 1e-6)
            heads.append(_bdot(attn.reshape(BM, 1, N), V.reshape(BM, N, H)))
        h0 = heads[0].reshape(BM, H)
        h1 = heads[1].reshape(BM, H)
        mts = jnp.tanh(jnp.dot(h0, m2sW0, preferred_element_type=_F32)
                       + jnp.dot(h1, m2sW1, preferred_element_type=_F32) + m2sb)

        Pv = jnp.dot(vf, U2v, preferred_element_type=_F32) + U2b
        Pe = jnp.dot(bf, U2e, preferred_element_type=_F32) + U2b
        Pcat = jnp.concatenate(
            [Pv.reshape(BM, N, H), Pe.reshape(BM, N, H)], axis=1)
        G = _bdot(oh, Pcat)
        L = _lrelu(G, 0.1)
        nei = _bdot(R, L).reshape(BM * N, H)
        main_self = _lrelu(
            jnp.dot(vf, U1v, preferred_element_type=_F32)
            + jnp.dot(nei, U1n, preferred_element_type=_F32) + U1b, 0.1)

        zm_in = (jnp.dot(main_self, zm1W, preferred_element_type=_F32) + zm1b
                 ).reshape(BM, N, H)
        zm_sup = jnp.dot(s2m, zm2W, preferred_element_type=_F32) + zm2b
        zm = jax.nn.sigmoid(zm_in + zm_sup.reshape(BM, 1, H))
        vf = ((1 - zm) * main_self.reshape(BM, N, H)
              + zm * s2m.reshape(BM, 1, H)).reshape(BM * N, H)
        zs = jax.nn.sigmoid(
            jnp.dot(ss, zs1W, preferred_element_type=_F32) + zs1b
            + jnp.dot(mts, zs2W, preferred_element_type=_F32) + zs2b)
        sf = (1 - zs) * ss + zs * mts

    vf_out[...] = vf.reshape(BM, N, H)
    sf_out[...] = sf.reshape(BM, 1, H)


def kernel(batch_size, atom_fea, bond_fea, d_anb, d_bnb, d_nbs_mask, d_mask, params):
    B = atom_fea.shape[0]
    flat = _prep_params(params)

    anb2 = d_anb.astype(jnp.int32).reshape(B, N * NBS, 1)
    bnb2 = d_bnb.astype(jnp.int32).reshape(B, N * NBS, 1)
    nmask2 = d_nbs_mask.reshape(B, 1, N * NBS)
    dmask2 = d_mask.reshape(B, 1, N)
    pat = jnp.repeat(jnp.eye(N, dtype=jnp.float32), NBS, axis=1
                     ).reshape(1, N, N * NBS)

    def rep(shape):
        nd = len(shape)
        return pl.BlockSpec(shape, lambda i, _n=nd: (0,) * _n)

    in_specs = [
        pl.BlockSpec((BM, N, ATOM_FDIM), lambda i: (i, 0, 0)),
        pl.BlockSpec((BM, N, BOND_FDIM), lambda i: (i, 0, 0)),
        pl.BlockSpec((BM, N * NBS, 1), lambda i: (i, 0, 0)),
        pl.BlockSpec((BM, N * NBS, 1), lambda i: (i, 0, 0)),
        pl.BlockSpec((BM, 1, N * NBS), lambda i: (i, 0, 0)),
        pl.BlockSpec((BM, 1, N), lambda i: (i, 0, 0)),
        rep((1, N, N * NBS)),
    ] + [rep(a.shape) for a in flat]

    out_shape = (
        jax.ShapeDtypeStruct((B, N, H), jnp.float32),
        jax.ShapeDtypeStruct((B, 1, H), jnp.float32),
    )
    out_specs = (
        pl.BlockSpec((BM, N, H), lambda i: (i, 0, 0)),
        pl.BlockSpec((BM, 1, H), lambda i: (i, 0, 0)),
    )
    vf, sf = pl.pallas_call(
        _body,
        grid=(B // BM,),
        in_specs=in_specs,
        out_specs=out_specs,
        out_shape=out_shape,
    )(atom_fea, bond_fea, anb2, bnb2, nmask2, dmask2, pat, *flat)
    return vf, sf
